# Initial kernel scaffold; baseline (speedup 1.0000x reference)
import jax
import jax.numpy as jnp
from jax.experimental import pallas as pl


def kernel(x):
    raise NotImplementedError("write your pallas kernel here")


# speedup vs baseline: 1.8519x; 1.8519x over previous
import functools

import jax
import jax.numpy as jnp
from jax.experimental import pallas as pl

N_TOP = 10
N_BOTTOM = 10
ROWS_PER_BLOCK = 8
N_COLS = 32768


def _extreme_block(x_ref, o_ref):
    x = x_ref[...]
    iota = jax.lax.broadcasted_iota(jnp.int32, x.shape, 1)
    big = jnp.int32(2**30)

    cur = x
    tops = []
    for _ in range(N_TOP):
        m = jnp.max(cur, axis=1, keepdims=True)
        tops.append(m)
        sel = jnp.where(cur == m, iota, big)
        jm = jnp.min(sel, axis=1, keepdims=True)
        cur = jnp.where(iota == jm, -jnp.inf, cur)

    cur = x
    bots = []
    for _ in range(N_BOTTOM):
        m = jnp.min(cur, axis=1, keepdims=True)
        bots.append(m)
        sel = jnp.where(cur == m, iota, big)
        jm = jnp.min(sel, axis=1, keepdims=True)
        cur = jnp.where(iota == jm, jnp.inf, cur)

    o_ref[...] = jnp.concatenate(tops + bots, axis=1)


@jax.jit
def kernel(x):
    n_rows = x.shape[0]
    grid = (n_rows // ROWS_PER_BLOCK,)
    return pl.pallas_call(
        _extreme_block,
        grid=grid,
        in_specs=[pl.BlockSpec((ROWS_PER_BLOCK, N_COLS), lambda i: (i, 0))],
        out_specs=pl.BlockSpec((ROWS_PER_BLOCK, N_TOP + N_BOTTOM), lambda i: (i, 0)),
        out_shape=jax.ShapeDtypeStruct((n_rows, N_TOP + N_BOTTOM), x.dtype),
    )(x)


# speedup vs baseline: 9.8235x; 5.3047x over previous
import jax
import jax.numpy as jnp
from jax.experimental import pallas as pl

N_TOP = 10
N_BOTTOM = 10
ROWS_PER_BLOCK = 8
N_COLS = 32768
GROUP = 16
LANES = 128


def _oems_pairs(n):
    pairs = []

    def merge(lo, m, r):
        step = r * 2
        if step < m:
            merge(lo, m, step)
            merge(lo + r, m, step)
            for i in range(lo + r, lo + m - r, step):
                pairs.append((i, i + r))
        else:
            pairs.append((lo, lo + r))

    def sort(lo, m):
        if m > 1:
            h = m // 2
            sort(lo, h)
            sort(lo + h, h)
            merge(lo, m, 1)

    sort(0, n)
    return pairs


_SORT16 = _oems_pairs(GROUP)


def _sort_group_asc(vals):
    vals = list(vals)
    for i, j in _SORT16:
        a, b = vals[i], vals[j]
        vals[i] = jnp.minimum(a, b)
        vals[j] = jnp.maximum(a, b)
    return vals


def _bitonic_cleanup(w, asc):
    for d in (8, 4, 2, 1):
        for i in range(16):
            if (i & d) == 0 and i + d < 16:
                a, b = w[i], w[i + d]
                if b is None:
                    continue
                if a is None:
                    w[i], w[i + d] = b, None
                    continue
                if asc:
                    w[i], w[i + d] = jnp.minimum(a, b), jnp.maximum(a, b)
                else:
                    w[i], w[i + d] = jnp.maximum(a, b), jnp.minimum(a, b)
    return w


def _merge_top(a, b):
    k = N_TOP
    m = [jnp.maximum(a[i], b[k - 1 - i]) for i in range(k)]
    w = _bitonic_cleanup(m + [None] * (16 - k), asc=True)
    return [w[k - 1 - i] for i in range(k)]


def _merge_bot(a, b):
    k = N_BOTTOM
    m = [jnp.minimum(a[i], b[k - 1 - i]) for i in range(k)]
    w = _bitonic_cleanup(m + [None] * (16 - k), asc=False)
    return [w[k - 1 - i] for i in range(k)]


def _extreme_block(x_ref, o_ref):
    n_groups = N_COLS // (GROUP * LANES)
    tops = None
    bots = None
    for g in range(n_groups):
        base = g * GROUP * LANES
        grp = [x_ref[:, base + t * LANES:base + (t + 1) * LANES]
               for t in range(GROUP)]
        s = _sort_group_asc(grp)
        g_top = [s[GROUP - 1 - i] for i in range(N_TOP)]
        g_bot = s[:N_BOTTOM]
        if tops is None:
            tops, bots = g_top, g_bot
        else:
            tops = _merge_top(tops, g_top)
            bots = _merge_bot(bots, g_bot)

    width = LANES
    while width > 1:
        half = width // 2
        t_lo = [t[:, :half] for t in tops]
        t_hi = [t[:, half:width] for t in tops]
        tops = _merge_top(t_lo, t_hi)
        b_lo = [b[:, :half] for b in bots]
        b_hi = [b[:, half:width] for b in bots]
        bots = _merge_bot(b_lo, b_hi)
        width = half

    o_ref[...] = jnp.concatenate(tops + bots, axis=1)


@jax.jit
def kernel(x):
    n_rows = x.shape[0]
    grid = (n_rows // ROWS_PER_BLOCK,)
    return pl.pallas_call(
        _extreme_block,
        grid=grid,
        in_specs=[pl.BlockSpec((ROWS_PER_BLOCK, N_COLS), lambda i: (i, 0))],
        out_specs=pl.BlockSpec((ROWS_PER_BLOCK, N_TOP + N_BOTTOM), lambda i: (i, 0)),
        out_shape=jax.ShapeDtypeStruct((n_rows, N_TOP + N_BOTTOM), x.dtype),
    )(x)
